# baseline (speedup 1.0000x reference)
import jax
import jax.numpy as jnp
from jax.experimental import pallas as pl
from jax.experimental.pallas import tpu as pltpu

N = 10000
NFEAT = 128
NHID = 128
CLASSES = 64

BI = 200
NI = N // BI


def _gcn_kernel(adj_ref, x_ref, w0_ref, w1_ref, w2_ref, out_ref,
                h1_ref, h2_ref, z_ref):
    l = pl.program_id(0)
    i = pl.program_id(1)

    @pl.when(i == 0)
    def _():
        @pl.when(l == 0)
        def _():
            z_ref[...] = jnp.dot(x_ref[...], w0_ref[...],
                                 preferred_element_type=jnp.float32)

        @pl.when(l == 1)
        def _():
            z_ref[...] = jnp.dot(h1_ref[...], w1_ref[...],
                                 preferred_element_type=jnp.float32)

        @pl.when(l == 2)
        def _():
            z_ref[:, :CLASSES] = jnp.dot(h2_ref[...], w2_ref[...],
                                         preferred_element_type=jnp.float32)

    row = jnp.dot(adj_ref[...], z_ref[...], preferred_element_type=jnp.float32)

    @pl.when(l == 0)
    def _():
        h1_ref[pl.ds(i * BI, BI), :] = jnp.maximum(row, 0.0)

    @pl.when(l == 1)
    def _():
        h2_ref[pl.ds(i * BI, BI), :] = jnp.maximum(row, 0.0)

    @pl.when(l == 2)
    def _():
        out_ref[...] = row[:, :CLASSES]


@jax.jit
def kernel(adj, x, W0, W1, W2):
    grid = (3, NI)
    return pl.pallas_call(
        _gcn_kernel,
        grid=grid,
        in_specs=[
            pl.BlockSpec((BI, N), lambda l, i: (i, 0)),
            pl.BlockSpec((N, NFEAT), lambda l, i: (0, 0)),
            pl.BlockSpec((NFEAT, NHID), lambda l, i: (0, 0)),
            pl.BlockSpec((NHID, NHID), lambda l, i: (0, 0)),
            pl.BlockSpec((NHID, CLASSES), lambda l, i: (0, 0)),
        ],
        out_specs=pl.BlockSpec((BI, CLASSES), lambda l, i: (i, 0)),
        out_shape=jax.ShapeDtypeStruct((N, CLASSES), jnp.float32),
        scratch_shapes=[
            pltpu.VMEM((N, NHID), jnp.float32),
            pltpu.VMEM((N, NHID), jnp.float32),
            pltpu.VMEM((N, NHID), jnp.float32),
        ],
        compiler_params=pltpu.CompilerParams(
            dimension_semantics=("arbitrary", "arbitrary"),
        ),
    )(adj, x, W0, W1, W2)
